# baseline (speedup 1.0000x reference)
import functools

import jax
import jax.numpy as jnp
from jax import lax
from jax.experimental import pallas as pl
from jax.experimental.pallas import tpu as pltpu
from jax.experimental.pallas import tpu_sc as plsc

L = 2
EPS = 0.0
AGGRS = ((1, 1, 1), (1, 1, 2), (1, 2, 2), (2, 1, 1), (2, 1, 2), (2, 2, 2))

NC = 2
NS = 16
NW = NC * NS


def _mesh():
    return plsc.VectorSubcoreMesh(core_axis_name="c", subcore_axis_name="s")


def _rowmul(dst_ref, a_ref, b_ref, nrows, groups, extra=None):
    def body(r, carry):
        for g in range(groups):
            sl = pl.ds(g * 16, 16)
            v = a_ref[r, sl] * b_ref[r, sl]
            if extra is not None:
                v = v + extra[r, sl]
            dst_ref[r, sl] = v
        return carry
    lax.fori_loop(0, nrows, body, 0)


def _stripes(total):
    q = (total // NS) // 8 * 8
    return q, total - NS * q

def _sc_gather_rows(table, idx):
    R, C = table.shape
    rows_per_w = R // NW
    GB = 400
    nb = rows_per_w // GB
    assert rows_per_w % GB == 0, (R, rows_per_w)

    @functools.partial(
        pl.kernel, mesh=_mesh(),
        out_type=jax.ShapeDtypeStruct((R, C), jnp.float32),
        scratch_types=[
            pltpu.VMEM((GB,), jnp.int32),
            pltpu.VMEM((GB, C), jnp.float32),
            pltpu.SemaphoreType.DMA,
        ],
    )
    def k(table_hbm, idx_hbm, out_hbm, idx_v, rows_v, sem):
        wid = lax.axis_index("s") * NC + lax.axis_index("c")

        def body(b, carry):
            base = wid * rows_per_w + b * GB
            pltpu.sync_copy(idx_hbm.at[pl.ds(base, GB)], idx_v)
            pltpu.async_copy(table_hbm.at[idx_v], rows_v, sem).wait()
            pltpu.sync_copy(rows_v, out_hbm.at[pl.ds(base, GB)])
            return carry

        lax.fori_loop(0, nb, body, 0)

    return k(table, idx.astype(jnp.int32))


def _sc_node_agg(a1, a2, ei1, ei2):
    E, C = a1.shape
    N = 10000
    GB = 200
    rows_per_w = E // NW
    nb = rows_per_w // GB
    stripe = N // NS

    q, tail = _stripes(N)

    @functools.partial(
        pl.kernel, mesh=_mesh(),
        out_type=jax.ShapeDtypeStruct((2, N, C), jnp.float32),
        scratch_types=[
            pltpu.VMEM_SHARED((N, C), jnp.float32),
            pltpu.VMEM((GB,), jnp.int32),
            pltpu.VMEM((GB, C), jnp.float32),
            pltpu.VMEM((104, C), jnp.float32),
            pltpu.SemaphoreType.DMA,
        ],
    )
    def k(a1_hbm, a2_hbm, e10_hbm, e11_hbm, e20_hbm, e21_hbm, p_hbm, acc_sp,
          idx_v, rows_v, zbuf, sem):
        c = lax.axis_index("c")
        s = lax.axis_index("s")
        wid = s * NC + c

        def zb(r, carry):
            for g in range(C // 16):
                zbuf[r, pl.ds(g * 16, 16)] = jnp.zeros((16,), jnp.float32)
            return carry
        lax.fori_loop(0, 104, zb, 0)
        for z in range(q // 104):
            pltpu.sync_copy(zbuf, acc_sp.at[pl.ds(s * q + z * 104, 104)])

        @pl.when(s == NS - 1)
        def _():
            pltpu.sync_copy(zbuf.at[pl.ds(0, tail)],
                            acc_sp.at[pl.ds(NS * q, tail)])
        plsc.subcore_barrier()

        for tab_hbm, eir_hbm in ((a1_hbm, e10_hbm), (a1_hbm, e11_hbm),
                                 (a2_hbm, e20_hbm), (a2_hbm, e21_hbm)):
            def body(b, carry):
                base = wid * rows_per_w + b * GB
                pltpu.sync_copy(eir_hbm.at[pl.ds(base, GB)], idx_v)
                pltpu.sync_copy(tab_hbm.at[pl.ds(base, GB)], rows_v)
                pltpu.sync_copy(rows_v, acc_sp.at[idx_v], add=True)
                return carry
            lax.fori_loop(0, nb, body, 0)

        plsc.subcore_barrier()
        pltpu.sync_copy(acc_sp.at[pl.ds(s * q, q)],
                        p_hbm.at[c, pl.ds(s * q, q)])

        @pl.when(s == NS - 1)
        def _():
            pltpu.sync_copy(acc_sp.at[pl.ds(NS * q, tail)],
                            p_hbm.at[c, pl.ds(NS * q, tail)])

    ei1 = ei1.astype(jnp.int32)
    ei2 = ei2.astype(jnp.int32)
    return k(a1, a2, ei1[0], ei1[1], ei2[0], ei2[1])


def _sc_edge_base(attr, a0, ei):
    E, C = attr.shape
    N, _ = a0.shape
    GB = 80
    rows_per_w = E // NW
    nb = rows_per_w // GB
    q, tail = _stripes(N)

    @functools.partial(
        pl.kernel, mesh=_mesh(),
        out_type=jax.ShapeDtypeStruct((E, C), jnp.float32),
        scratch_types=[
            pltpu.VMEM_SHARED((N, C), jnp.float32),
            pltpu.VMEM((GB,), jnp.int32),
            pltpu.VMEM((GB,), jnp.int32),
            pltpu.VMEM((GB, C), jnp.float32),
            pltpu.VMEM((GB, C), jnp.float32),
            pltpu.VMEM((GB, C), jnp.float32),
            pltpu.SemaphoreType.DMA,
            pltpu.SemaphoreType.DMA,
        ],
    )
    def k(attr_hbm, a0_hbm, e0_hbm, e1_hbm, out_hbm, a0_sp, i0_v, i1_v, ga_v,
          gb_v, av_v, sem0, sem1):
        c = lax.axis_index("c")
        s = lax.axis_index("s")
        wid = s * NC + c

        pltpu.sync_copy(a0_hbm.at[pl.ds(s * q, q)], a0_sp.at[pl.ds(s * q, q)])

        @pl.when(s == NS - 1)
        def _():
            pltpu.sync_copy(a0_hbm.at[pl.ds(NS * q, tail)],
                            a0_sp.at[pl.ds(NS * q, tail)])
        plsc.subcore_barrier()

        def body(b, carry):
            base = wid * rows_per_w + b * GB
            pltpu.sync_copy(e0_hbm.at[pl.ds(base, GB)], i0_v)
            pltpu.sync_copy(e1_hbm.at[pl.ds(base, GB)], i1_v)
            cp0 = pltpu.async_copy(a0_sp.at[i0_v], ga_v, sem0)
            cp1 = pltpu.async_copy(a0_sp.at[i1_v], gb_v, sem1)
            pltpu.sync_copy(attr_hbm.at[pl.ds(base, GB)], av_v)
            cp0.wait()
            cp1.wait()
            _rowmul(av_v, ga_v, gb_v, GB, C // 16, extra=av_v)
            pltpu.sync_copy(av_v, out_hbm.at[pl.ds(base, GB)])
            return carry

        lax.fori_loop(0, nb, body, 0)

    ei = ei.astype(jnp.int32)
    return k(attr, a0, ei[0], ei[1])


def _sc_tri_acc(tris_jk, tab1, tab2, base):
    E, C = base.shape
    T = tris_jk[0][0].shape[1]
    CH = E // 32
    stripe = CH // NS
    IB = 800
    tpt = T // NS
    nib = tpt // IB
    nvs = IB // 16
    CAP = 128
    FTH = CAP - 16
    assert T % NS == 0 and (T // NS) % IB == 0 and IB % 16 == 0

    tabs = {1: 0, 2: 1}
    q, tail = _stripes(CH)

    @functools.partial(
        pl.kernel, mesh=_mesh(),
        compiler_params=pltpu.CompilerParams(needs_layout_passes=False),
        out_type=jax.ShapeDtypeStruct((E, C), jnp.float32),
        scratch_types=[
            pltpu.VMEM_SHARED((CH + 8, C), jnp.float32),
            pltpu.VMEM((IB,), jnp.int32),
            pltpu.VMEM((IB,), jnp.int32),
            pltpu.VMEM((IB,), jnp.int32),
            pltpu.VMEM((CAP,), jnp.int32),
            pltpu.VMEM((CAP,), jnp.int32),
            pltpu.VMEM((CAP,), jnp.int32),
            pltpu.VMEM((CAP, C), jnp.float32),
            pltpu.VMEM((CAP, C), jnp.float32),
            pltpu.SemaphoreType.DMA,
            pltpu.SemaphoreType.DMA,
        ],
    )
    def k(t00, t01, t02, t10, t11, t12, t20, t21, t22, tA_hbm, tB_hbm,
          base_hbm, out_hbm, acc_sp, is0, is1, is2, cb0, cb1, cb2, gj_v,
          gk_v, sem0, sem1):
        c = lax.axis_index("c")
        s = lax.axis_index("s")
        tri_hbms = ((t00, t01, t02), (t10, t11, t12), (t20, t21, t22))
        tab_hbms = (tA_hbm, tB_hbm)

        lanes = lax.iota(jnp.int32, 16)

        def reset_cbufs():
            for g in range(CAP // 16):
                sl = pl.ds(g * 16, 16)
                cb0[sl] = jnp.full((16,), CH, jnp.int32)
                cb1[sl] = lanes + g * 16
                cb2[sl] = lanes + g * 16

        def chunk_body(ch_i, carry0):
            chunk = c * 16 + ch_i
            lo = chunk * CH

            pltpu.sync_copy(base_hbm.at[pl.ds(lo + s * q, q)],
                            acc_sp.at[pl.ds(s * q, q)])

            @pl.when(s == NS - 1)
            def _():
                pltpu.sync_copy(base_hbm.at[pl.ds(lo + NS * q, tail)],
                                acc_sp.at[pl.ds(NS * q, tail)])
            plsc.subcore_barrier()

            for tri_idx, (tri, j, kk) in enumerate(tris_jk):
                trow0, trow1, trow2 = tri_hbms[tri_idx]
                tj_hbm = tab_hbms[tabs[j]]
                tk_hbm = tab_hbms[tabs[kk]]

                def flush():
                    cpj = pltpu.async_copy(tj_hbm.at[cb1], gj_v, sem0)
                    cpk = pltpu.async_copy(tk_hbm.at[cb2], gk_v, sem1)
                    cpj.wait()
                    cpk.wait()
                    _rowmul(gj_v, gj_v, gk_v, CAP, C // 16)
                    pltpu.sync_copy(gj_v, acc_sp.at[cb0], add=True)
                    reset_cbufs()

                def batch_body(b, cnt):
                    base_t = s * tpt + b * IB
                    pltpu.sync_copy(trow0.at[pl.ds(base_t, IB)], is0)
                    pltpu.sync_copy(trow1.at[pl.ds(base_t, IB)], is1)
                    pltpu.sync_copy(trow2.at[pl.ds(base_t, IB)], is2)

                    def vec_body(v, cnt):
                        sl = pl.ds(v * 16, 16)
                        t0 = is0[sl]
                        m = (t0 >= lo) & (t0 < lo + CH)
                        nm = jnp.sum(m.astype(jnp.int32))

                        def compact(cnt):
                            need = cnt > FTH
                            @pl.when(need)
                            def _():
                                flush()
                            cnt = jnp.where(need, 0, cnt)
                            mi = m.astype(jnp.int32)
                            pos = cnt + plsc.cumsum(mi) - 1
                            plsc.store_scatter(cb0, [pos], t0 - lo, mask=m)
                            plsc.store_scatter(cb1, [pos], is1[sl], mask=m)
                            plsc.store_scatter(cb2, [pos], is2[sl], mask=m)
                            return cnt + nm

                        return lax.cond(nm > 0, compact, lambda cnt: cnt, cnt)

                    return lax.fori_loop(0, nvs, vec_body, cnt)

                reset_cbufs()
                cnt = lax.fori_loop(0, nib, batch_body, jnp.int32(0))

                @pl.when(cnt > 0)
                def _():
                    flush()

            plsc.subcore_barrier()
            pltpu.sync_copy(acc_sp.at[pl.ds(s * q, q)],
                            out_hbm.at[pl.ds(lo + s * q, q)])

            @pl.when(s == NS - 1)
            def _():
                pltpu.sync_copy(acc_sp.at[pl.ds(NS * q, tail)],
                                out_hbm.at[pl.ds(lo + NS * q, tail)])
            plsc.subcore_barrier()
            return carry0

        lax.fori_loop(0, 16, chunk_body, 0)

    t = [x[0].astype(jnp.int32) for x in tris_jk]
    return k(t[0][0], t[0][1], t[0][2], t[1][0], t[1][1], t[1][2],
             t[2][0], t[2][1], t[2][2], tab1, tab2, base)


def _blk(R):
    return 512 if R % 512 == 0 else 400


def _matmul_stats(xs, w, b, coef):
    R, C = xs[0].shape
    BLK = _blk(R)
    grid = R // BLK
    nx = len(xs)

    def body(*refs):
        xa_refs = refs[:nx]
        w_ref, b_ref, y_ref, s1_ref, s2_ref = refs[nx:]
        i = pl.program_id(0)
        x = xa_refs[0][...]
        for r in xa_refs[1:]:
            x = x + r[...]
        if coef != 1.0:
            x = x * coef
        y = jnp.dot(x, w_ref[...], preferred_element_type=jnp.float32)
        y = y + b_ref[...]
        y_ref[...] = y
        ps1 = jnp.broadcast_to(jnp.sum(y, axis=0, keepdims=True), s1_ref.shape)
        ps2 = jnp.broadcast_to(jnp.sum(y * y, axis=0, keepdims=True),
                               s2_ref.shape)

        @pl.when(i == 0)
        def _():
            s1_ref[...] = jnp.zeros_like(s1_ref)
            s2_ref[...] = jnp.zeros_like(s2_ref)

        s1_ref[...] += ps1
        s2_ref[...] += ps2

    return pl.pallas_call(
        body,
        grid=(grid,),
        in_specs=[pl.BlockSpec((BLK, C), lambda i: (i, 0)) for _ in range(nx)]
        + [pl.BlockSpec((C, C), lambda i: (0, 0)),
           pl.BlockSpec((C,), lambda i: (0,))],
        out_specs=[pl.BlockSpec((BLK, C), lambda i: (i, 0)),
                   pl.BlockSpec((8, C), lambda i: (0, 0)),
                   pl.BlockSpec((8, C), lambda i: (0, 0))],
        out_shape=[jax.ShapeDtypeStruct((R, C), jnp.float32),
                   jax.ShapeDtypeStruct((8, C), jnp.float32),
                   jax.ShapeDtypeStruct((8, C), jnp.float32)],
    )(*xs, w, b)


def _bn_relu_kernel(y_ref, s1_ref, s2_ref, g_ref, be_ref, o_ref, *, R):
    mu = s1_ref[0:1, :] / R
    var = s2_ref[0:1, :] / R - mu * mu
    o_ref[...] = jnp.maximum(
        (y_ref[...] - mu) * lax.rsqrt(var + 1e-5) * g_ref[...] + be_ref[...],
        0.0)


def _bn_relu(y, s1, s2, g, be):
    R, C = y.shape
    BLK = _blk(R)
    grid = R // BLK
    return pl.pallas_call(
        functools.partial(_bn_relu_kernel, R=float(R)),
        grid=(grid,),
        in_specs=[pl.BlockSpec((BLK, C), lambda i: (i, 0)),
                  pl.BlockSpec((8, C), lambda i: (0, 0)),
                  pl.BlockSpec((8, C), lambda i: (0, 0)),
                  pl.BlockSpec((C,), lambda i: (0,)),
                  pl.BlockSpec((C,), lambda i: (0,))],
        out_specs=pl.BlockSpec((BLK, C), lambda i: (i, 0)),
        out_shape=jax.ShapeDtypeStruct((R, C), jnp.float32),
    )(y, s1, s2, g, be)


def _mm_kernel(x_ref, w_ref, b_ref, o_ref):
    o_ref[...] = jnp.dot(x_ref[...], w_ref[...],
                         preferred_element_type=jnp.float32) + b_ref[...]


def _mm(x, w, b):
    R, C = x.shape
    BLK = _blk(R)
    grid = R // BLK
    return pl.pallas_call(
        _mm_kernel,
        grid=(grid,),
        in_specs=[pl.BlockSpec((BLK, C), lambda i: (i, 0)),
                  pl.BlockSpec((C, C), lambda i: (0, 0)),
                  pl.BlockSpec((C,), lambda i: (0,))],
        out_specs=pl.BlockSpec((BLK, C), lambda i: (i, 0)),
        out_shape=jax.ShapeDtypeStruct((R, C), jnp.float32),
    )(x, w, b)


def kernel(a0, a1, a2, ei1, ei2, tri_111, tri_112, tri_122, tri_211, tri_212,
           tri_222, inv1, inv2, W_gnn, b_gnn, gamma, beta, W_out, b_out):
    tris = {(1, 1, 1): tri_111, (1, 1, 2): tri_112, (1, 2, 2): tri_122,
            (2, 1, 1): tri_211, (2, 1, 2): tri_212, (2, 2, 2): tri_222}
    eis = [None, ei1, ei2]
    invs = [None, inv1, inv2]
    attrs = [a0, a1, a2]
    for layer in range(L):
        p = _sc_node_agg(attrs[1], attrs[2], ei1, ei2)
        hraw = [None, None, None]
        for l in (1, 2):
            base = _sc_edge_base(attrs[l], attrs[0], eis[l])
            tjk = [(tris[(l, 1, 1)], 1, 1), (tris[(l, 1, 2)], 1, 2),
                   (tris[(l, 2, 2)], 2, 2)]
            hraw[l] = _sc_tri_acc(tjk, attrs[1], attrs[2], base)
        new_attrs = []
        for l in range(3):
            if l > 0:
                hg = _sc_gather_rows(hraw[l], invs[l])
                y, s1, s2 = _matmul_stats([hraw[l], hg], W_gnn[layer, l],
                                          b_gnn[layer, l], 0.5)
            else:
                y, s1, s2 = _matmul_stats([attrs[0], p[0], p[1]], W_gnn[layer, l],
                                          b_gnn[layer, l], 1.0)
            new_attrs.append(_bn_relu(y, s1, s2, gamma[layer, l],
                                      beta[layer, l]))
        attrs = new_attrs
    return tuple(_mm(attrs[l], W_out[l], b_out[l]) for l in range(3))


# speedup vs baseline: 1.3204x; 1.3204x over previous
import functools

import jax
import jax.numpy as jnp
from jax import lax
from jax.experimental import pallas as pl
from jax.experimental.pallas import tpu as pltpu
from jax.experimental.pallas import tpu_sc as plsc

L = 2
EPS = 0.0
AGGRS = ((1, 1, 1), (1, 1, 2), (1, 2, 2), (2, 1, 1), (2, 1, 2), (2, 2, 2))

NC = 2
NS = 16
NW = NC * NS


def _mesh():
    return plsc.VectorSubcoreMesh(core_axis_name="c", subcore_axis_name="s")


def _rowmul(dst_ref, a_ref, b_ref, nrows, groups, extra=None):
    def body(r, carry):
        for g in range(groups):
            sl = pl.ds(g * 16, 16)
            v = a_ref[r, sl] * b_ref[r, sl]
            if extra is not None:
                v = v + extra[r, sl]
            dst_ref[r, sl] = v
        return carry
    lax.fori_loop(0, nrows, body, 0)


def _stripes(total):
    q = (total // NS) // 8 * 8
    return q, total - NS * q

def _sc_gather_rows(table, idx):
    R, C = table.shape
    rows_per_w = R // NW
    GB = 400
    nb = rows_per_w // GB
    assert rows_per_w % GB == 0, (R, rows_per_w)

    @functools.partial(
        pl.kernel, mesh=_mesh(),
        out_type=jax.ShapeDtypeStruct((R, C), jnp.float32),
        scratch_types=[
            pltpu.VMEM((GB,), jnp.int32),
            pltpu.VMEM((GB, C), jnp.float32),
            pltpu.SemaphoreType.DMA,
        ],
    )
    def k(table_hbm, idx_hbm, out_hbm, idx_v, rows_v, sem):
        wid = lax.axis_index("s") * NC + lax.axis_index("c")

        def body(b, carry):
            base = wid * rows_per_w + b * GB
            pltpu.sync_copy(idx_hbm.at[pl.ds(base, GB)], idx_v)
            pltpu.async_copy(table_hbm.at[idx_v], rows_v, sem).wait()
            pltpu.sync_copy(rows_v, out_hbm.at[pl.ds(base, GB)])
            return carry

        lax.fori_loop(0, nb, body, 0)

    return k(table, idx.astype(jnp.int32))


def _sc_node_agg(a1, a2, ei1, ei2):
    E, C = a1.shape
    N = 10000
    GB = 200
    rows_per_w = E // NW
    nb = rows_per_w // GB
    stripe = N // NS

    q, tail = _stripes(N)

    @functools.partial(
        pl.kernel, mesh=_mesh(),
        out_type=jax.ShapeDtypeStruct((2, N, C), jnp.float32),
        scratch_types=[
            pltpu.VMEM_SHARED((N, C), jnp.float32),
            pltpu.VMEM((GB,), jnp.int32),
            pltpu.VMEM((GB, C), jnp.float32),
            pltpu.VMEM((104, C), jnp.float32),
            pltpu.SemaphoreType.DMA,
        ],
    )
    def k(a1_hbm, a2_hbm, e10_hbm, e11_hbm, e20_hbm, e21_hbm, p_hbm, acc_sp,
          idx_v, rows_v, zbuf, sem):
        c = lax.axis_index("c")
        s = lax.axis_index("s")
        wid = s * NC + c

        def zb(r, carry):
            for g in range(C // 16):
                zbuf[r, pl.ds(g * 16, 16)] = jnp.zeros((16,), jnp.float32)
            return carry
        lax.fori_loop(0, 104, zb, 0)
        for z in range(q // 104):
            pltpu.sync_copy(zbuf, acc_sp.at[pl.ds(s * q + z * 104, 104)])

        @pl.when(s == NS - 1)
        def _():
            pltpu.sync_copy(zbuf.at[pl.ds(0, tail)],
                            acc_sp.at[pl.ds(NS * q, tail)])
        plsc.subcore_barrier()

        for tab_hbm, eir_hbm in ((a1_hbm, e10_hbm), (a1_hbm, e11_hbm),
                                 (a2_hbm, e20_hbm), (a2_hbm, e21_hbm)):
            def body(b, carry):
                base = wid * rows_per_w + b * GB
                pltpu.sync_copy(eir_hbm.at[pl.ds(base, GB)], idx_v)
                pltpu.sync_copy(tab_hbm.at[pl.ds(base, GB)], rows_v)
                pltpu.sync_copy(rows_v, acc_sp.at[idx_v], add=True)
                return carry
            lax.fori_loop(0, nb, body, 0)

        plsc.subcore_barrier()
        pltpu.sync_copy(acc_sp.at[pl.ds(s * q, q)],
                        p_hbm.at[c, pl.ds(s * q, q)])

        @pl.when(s == NS - 1)
        def _():
            pltpu.sync_copy(acc_sp.at[pl.ds(NS * q, tail)],
                            p_hbm.at[c, pl.ds(NS * q, tail)])

    ei1 = ei1.astype(jnp.int32)
    ei2 = ei2.astype(jnp.int32)
    return k(a1, a2, ei1[0], ei1[1], ei2[0], ei2[1])


def _sc_edge_base(attr, a0, ei):
    E, C = attr.shape
    N, _ = a0.shape
    GB = 80
    rows_per_w = E // NW
    nb = rows_per_w // GB
    q, tail = _stripes(N)

    @functools.partial(
        pl.kernel, mesh=_mesh(),
        out_type=jax.ShapeDtypeStruct((E, C), jnp.float32),
        scratch_types=[
            pltpu.VMEM_SHARED((N, C), jnp.float32),
            pltpu.VMEM((GB,), jnp.int32),
            pltpu.VMEM((GB,), jnp.int32),
            pltpu.VMEM((GB, C), jnp.float32),
            pltpu.VMEM((GB, C), jnp.float32),
            pltpu.VMEM((GB, C), jnp.float32),
            pltpu.SemaphoreType.DMA,
            pltpu.SemaphoreType.DMA,
        ],
    )
    def k(attr_hbm, a0_hbm, e0_hbm, e1_hbm, out_hbm, a0_sp, i0_v, i1_v, ga_v,
          gb_v, av_v, sem0, sem1):
        c = lax.axis_index("c")
        s = lax.axis_index("s")
        wid = s * NC + c

        pltpu.sync_copy(a0_hbm.at[pl.ds(s * q, q)], a0_sp.at[pl.ds(s * q, q)])

        @pl.when(s == NS - 1)
        def _():
            pltpu.sync_copy(a0_hbm.at[pl.ds(NS * q, tail)],
                            a0_sp.at[pl.ds(NS * q, tail)])
        plsc.subcore_barrier()

        def body(b, carry):
            base = wid * rows_per_w + b * GB
            pltpu.sync_copy(e0_hbm.at[pl.ds(base, GB)], i0_v)
            pltpu.sync_copy(e1_hbm.at[pl.ds(base, GB)], i1_v)
            cp0 = pltpu.async_copy(a0_sp.at[i0_v], ga_v, sem0)
            cp1 = pltpu.async_copy(a0_sp.at[i1_v], gb_v, sem1)
            pltpu.sync_copy(attr_hbm.at[pl.ds(base, GB)], av_v)
            cp0.wait()
            cp1.wait()
            _rowmul(av_v, ga_v, gb_v, GB, C // 16, extra=av_v)
            pltpu.sync_copy(av_v, out_hbm.at[pl.ds(base, GB)])
            return carry

        lax.fori_loop(0, nb, body, 0)

    ei = ei.astype(jnp.int32)
    return k(attr, a0, ei[0], ei[1])


def _sc_tri_acc(tris_jk, tab1, tab2, base):
    E, C = base.shape
    T = tris_jk[0][0].shape[1]
    CH = E // 32
    stripe = CH // NS
    IB = 4000
    tpt = T // NS
    nib = tpt // IB
    nvs = IB // 16
    CAP = 128
    FTH = CAP - 16
    assert T % NS == 0 and (T // NS) % IB == 0 and IB % 16 == 0

    tabs = {1: 0, 2: 1}
    q, tail = _stripes(CH)

    @functools.partial(
        pl.kernel, mesh=_mesh(),
        compiler_params=pltpu.CompilerParams(needs_layout_passes=False),
        out_type=jax.ShapeDtypeStruct((E, C), jnp.float32),
        scratch_types=[
            pltpu.VMEM_SHARED((CH + 8, C), jnp.float32),
            pltpu.VMEM((IB,), jnp.int32),
            pltpu.VMEM((IB,), jnp.int32),
            pltpu.VMEM((IB,), jnp.int32),
            pltpu.VMEM((CAP,), jnp.int32),
            pltpu.VMEM((CAP,), jnp.int32),
            pltpu.VMEM((CAP,), jnp.int32),
            pltpu.VMEM((CAP, C), jnp.float32),
            pltpu.VMEM((CAP, C), jnp.float32),
            pltpu.SemaphoreType.DMA,
            pltpu.SemaphoreType.DMA,
        ],
    )
    def k(t00, t01, t02, t10, t11, t12, t20, t21, t22, tA_hbm, tB_hbm,
          base_hbm, out_hbm, acc_sp, is0, is1, is2, cb0, cb1, cb2, gj_v,
          gk_v, sem0, sem1):
        c = lax.axis_index("c")
        s = lax.axis_index("s")
        tri_hbms = ((t00, t01, t02), (t10, t11, t12), (t20, t21, t22))
        tab_hbms = (tA_hbm, tB_hbm)

        lanes = lax.iota(jnp.int32, 16)

        def reset_cbufs():
            for g in range(CAP // 16):
                sl = pl.ds(g * 16, 16)
                cb0[sl] = jnp.full((16,), CH, jnp.int32)
                cb1[sl] = lanes + g * 16
                cb2[sl] = lanes + g * 16

        def chunk_body(ch_i, carry0):
            chunk = c * 16 + ch_i
            lo = chunk * CH

            pltpu.sync_copy(base_hbm.at[pl.ds(lo + s * q, q)],
                            acc_sp.at[pl.ds(s * q, q)])

            @pl.when(s == NS - 1)
            def _():
                pltpu.sync_copy(base_hbm.at[pl.ds(lo + NS * q, tail)],
                                acc_sp.at[pl.ds(NS * q, tail)])
            plsc.subcore_barrier()

            for tri_idx, (tri, j, kk) in enumerate(tris_jk):
                trow0, trow1, trow2 = tri_hbms[tri_idx]
                tj_hbm = tab_hbms[tabs[j]]
                tk_hbm = tab_hbms[tabs[kk]]

                def flush():
                    cpj = pltpu.async_copy(tj_hbm.at[cb1], gj_v, sem0)
                    cpk = pltpu.async_copy(tk_hbm.at[cb2], gk_v, sem1)
                    cpj.wait()
                    cpk.wait()
                    _rowmul(gj_v, gj_v, gk_v, CAP, C // 16)
                    pltpu.sync_copy(gj_v, acc_sp.at[cb0], add=True)
                    reset_cbufs()

                def batch_body(b, cnt):
                    base_t = s * tpt + b * IB
                    pltpu.sync_copy(trow0.at[pl.ds(base_t, IB)], is0)
                    pltpu.sync_copy(trow1.at[pl.ds(base_t, IB)], is1)
                    pltpu.sync_copy(trow2.at[pl.ds(base_t, IB)], is2)

                    def vec_body(v, cnt):
                        sl = pl.ds(v * 16, 16)
                        t0 = is0[sl]
                        m = (t0 >= lo) & (t0 < lo + CH)
                        nm = jnp.sum(m.astype(jnp.int32))

                        def compact(cnt):
                            need = cnt > FTH
                            @pl.when(need)
                            def _():
                                flush()
                            cnt = jnp.where(need, 0, cnt)
                            mi = m.astype(jnp.int32)
                            pos = cnt + plsc.cumsum(mi) - 1
                            plsc.store_scatter(cb0, [pos], t0 - lo, mask=m)
                            plsc.store_scatter(cb1, [pos], is1[sl], mask=m)
                            plsc.store_scatter(cb2, [pos], is2[sl], mask=m)
                            return cnt + nm

                        return lax.cond(nm > 0, compact, lambda cnt: cnt, cnt)

                    return lax.fori_loop(0, nvs, vec_body, cnt)

                reset_cbufs()
                cnt = lax.fori_loop(0, nib, batch_body, jnp.int32(0))

                @pl.when(cnt > 0)
                def _():
                    flush()

            plsc.subcore_barrier()
            pltpu.sync_copy(acc_sp.at[pl.ds(s * q, q)],
                            out_hbm.at[pl.ds(lo + s * q, q)])

            @pl.when(s == NS - 1)
            def _():
                pltpu.sync_copy(acc_sp.at[pl.ds(NS * q, tail)],
                                out_hbm.at[pl.ds(lo + NS * q, tail)])
            plsc.subcore_barrier()
            return carry0

        lax.fori_loop(0, 16, chunk_body, 0)

    t = [x[0].astype(jnp.int32) for x in tris_jk]
    return k(t[0][0], t[0][1], t[0][2], t[1][0], t[1][1], t[1][2],
             t[2][0], t[2][1], t[2][2], tab1, tab2, base)


def _blk(R):
    return 512 if R % 512 == 0 else 400


def _matmul_stats(xs, w, b, coef):
    R, C = xs[0].shape
    BLK = _blk(R)
    grid = R // BLK
    nx = len(xs)

    def body(*refs):
        xa_refs = refs[:nx]
        w_ref, b_ref, y_ref, s1_ref, s2_ref = refs[nx:]
        i = pl.program_id(0)
        x = xa_refs[0][...]
        for r in xa_refs[1:]:
            x = x + r[...]
        if coef != 1.0:
            x = x * coef
        y = jnp.dot(x, w_ref[...], preferred_element_type=jnp.float32)
        y = y + b_ref[...]
        y_ref[...] = y
        ps1 = jnp.broadcast_to(jnp.sum(y, axis=0, keepdims=True), s1_ref.shape)
        ps2 = jnp.broadcast_to(jnp.sum(y * y, axis=0, keepdims=True),
                               s2_ref.shape)

        @pl.when(i == 0)
        def _():
            s1_ref[...] = jnp.zeros_like(s1_ref)
            s2_ref[...] = jnp.zeros_like(s2_ref)

        s1_ref[...] += ps1
        s2_ref[...] += ps2

    return pl.pallas_call(
        body,
        grid=(grid,),
        in_specs=[pl.BlockSpec((BLK, C), lambda i: (i, 0)) for _ in range(nx)]
        + [pl.BlockSpec((C, C), lambda i: (0, 0)),
           pl.BlockSpec((C,), lambda i: (0,))],
        out_specs=[pl.BlockSpec((BLK, C), lambda i: (i, 0)),
                   pl.BlockSpec((8, C), lambda i: (0, 0)),
                   pl.BlockSpec((8, C), lambda i: (0, 0))],
        out_shape=[jax.ShapeDtypeStruct((R, C), jnp.float32),
                   jax.ShapeDtypeStruct((8, C), jnp.float32),
                   jax.ShapeDtypeStruct((8, C), jnp.float32)],
    )(*xs, w, b)


def _bn_relu_kernel(y_ref, s1_ref, s2_ref, g_ref, be_ref, o_ref, *, R):
    mu = s1_ref[0:1, :] / R
    var = s2_ref[0:1, :] / R - mu * mu
    o_ref[...] = jnp.maximum(
        (y_ref[...] - mu) * lax.rsqrt(var + 1e-5) * g_ref[...] + be_ref[...],
        0.0)


def _bn_relu(y, s1, s2, g, be):
    R, C = y.shape
    BLK = _blk(R)
    grid = R // BLK
    return pl.pallas_call(
        functools.partial(_bn_relu_kernel, R=float(R)),
        grid=(grid,),
        in_specs=[pl.BlockSpec((BLK, C), lambda i: (i, 0)),
                  pl.BlockSpec((8, C), lambda i: (0, 0)),
                  pl.BlockSpec((8, C), lambda i: (0, 0)),
                  pl.BlockSpec((C,), lambda i: (0,)),
                  pl.BlockSpec((C,), lambda i: (0,))],
        out_specs=pl.BlockSpec((BLK, C), lambda i: (i, 0)),
        out_shape=jax.ShapeDtypeStruct((R, C), jnp.float32),
    )(y, s1, s2, g, be)


def _mm_kernel(x_ref, w_ref, b_ref, o_ref):
    o_ref[...] = jnp.dot(x_ref[...], w_ref[...],
                         preferred_element_type=jnp.float32) + b_ref[...]


def _mm(x, w, b):
    R, C = x.shape
    BLK = _blk(R)
    grid = R // BLK
    return pl.pallas_call(
        _mm_kernel,
        grid=(grid,),
        in_specs=[pl.BlockSpec((BLK, C), lambda i: (i, 0)),
                  pl.BlockSpec((C, C), lambda i: (0, 0)),
                  pl.BlockSpec((C,), lambda i: (0,))],
        out_specs=pl.BlockSpec((BLK, C), lambda i: (i, 0)),
        out_shape=jax.ShapeDtypeStruct((R, C), jnp.float32),
    )(x, w, b)


def kernel(a0, a1, a2, ei1, ei2, tri_111, tri_112, tri_122, tri_211, tri_212,
           tri_222, inv1, inv2, W_gnn, b_gnn, gamma, beta, W_out, b_out):
    tris = {(1, 1, 1): tri_111, (1, 1, 2): tri_112, (1, 2, 2): tri_122,
            (2, 1, 1): tri_211, (2, 1, 2): tri_212, (2, 2, 2): tri_222}
    eis = [None, ei1, ei2]
    invs = [None, inv1, inv2]
    attrs = [a0, a1, a2]
    for layer in range(L):
        p = _sc_node_agg(attrs[1], attrs[2], ei1, ei2)
        hraw = [None, None, None]
        for l in (1, 2):
            base = _sc_edge_base(attrs[l], attrs[0], eis[l])
            tjk = [(tris[(l, 1, 1)], 1, 1), (tris[(l, 1, 2)], 1, 2),
                   (tris[(l, 2, 2)], 2, 2)]
            hraw[l] = _sc_tri_acc(tjk, attrs[1], attrs[2], base)
        new_attrs = []
        for l in range(3):
            if l > 0:
                hg = _sc_gather_rows(hraw[l], invs[l])
                y, s1, s2 = _matmul_stats([hraw[l], hg], W_gnn[layer, l],
                                          b_gnn[layer, l], 0.5)
            else:
                y, s1, s2 = _matmul_stats([attrs[0], p[0], p[1]], W_gnn[layer, l],
                                          b_gnn[layer, l], 1.0)
            new_attrs.append(_bn_relu(y, s1, s2, gamma[layer, l],
                                      beta[layer, l]))
        attrs = new_attrs
    return tuple(_mm(attrs[l], W_out[l], b_out[l]) for l in range(3))


# speedup vs baseline: 1.3936x; 1.0554x over previous
import functools

import jax
import jax.numpy as jnp
from jax import lax
from jax.experimental import pallas as pl
from jax.experimental.pallas import tpu as pltpu
from jax.experimental.pallas import tpu_sc as plsc

L = 2
EPS = 0.0
AGGRS = ((1, 1, 1), (1, 1, 2), (1, 2, 2), (2, 1, 1), (2, 1, 2), (2, 2, 2))

NC = 2
NS = 16
NW = NC * NS


def _mesh():
    return plsc.VectorSubcoreMesh(core_axis_name="c", subcore_axis_name="s")


def _rowmul(dst_ref, a_ref, b_ref, nrows, groups, extra=None):
    def body(r, carry):
        for g in range(groups):
            sl = pl.ds(g * 16, 16)
            v = a_ref[r, sl] * b_ref[r, sl]
            if extra is not None:
                v = v + extra[r, sl]
            dst_ref[r, sl] = v
        return carry
    lax.fori_loop(0, nrows, body, 0)


def _stripes(total):
    q = (total // NS) // 8 * 8
    return q, total - NS * q

def _sc_gather_rows(table, idx):
    R, C = table.shape
    rows_per_w = R // NW
    GB = 400
    nb = rows_per_w // GB
    assert rows_per_w % GB == 0, (R, rows_per_w)

    @functools.partial(
        pl.kernel, mesh=_mesh(),
        out_type=jax.ShapeDtypeStruct((R, C), jnp.float32),
        scratch_types=[
            pltpu.VMEM((GB,), jnp.int32),
            pltpu.VMEM((GB, C), jnp.float32),
            pltpu.SemaphoreType.DMA,
        ],
    )
    def k(table_hbm, idx_hbm, out_hbm, idx_v, rows_v, sem):
        wid = lax.axis_index("s") * NC + lax.axis_index("c")

        def body(b, carry):
            base = wid * rows_per_w + b * GB
            pltpu.sync_copy(idx_hbm.at[pl.ds(base, GB)], idx_v)
            pltpu.async_copy(table_hbm.at[idx_v], rows_v, sem).wait()
            pltpu.sync_copy(rows_v, out_hbm.at[pl.ds(base, GB)])
            return carry

        lax.fori_loop(0, nb, body, 0)

    return k(table, idx.astype(jnp.int32))


def _sc_node_agg(a1, a2, ei1, ei2):
    E, C = a1.shape
    N = 10000
    GB = 200
    rows_per_w = E // NW
    nb = rows_per_w // GB
    stripe = N // NS

    q, tail = _stripes(N)

    @functools.partial(
        pl.kernel, mesh=_mesh(),
        out_type=jax.ShapeDtypeStruct((2, N, C), jnp.float32),
        scratch_types=[
            pltpu.VMEM_SHARED((N, C), jnp.float32),
            pltpu.VMEM((GB,), jnp.int32),
            pltpu.VMEM((GB, C), jnp.float32),
            pltpu.VMEM((104, C), jnp.float32),
            pltpu.SemaphoreType.DMA,
        ],
    )
    def k(a1_hbm, a2_hbm, e10_hbm, e11_hbm, e20_hbm, e21_hbm, p_hbm, acc_sp,
          idx_v, rows_v, zbuf, sem):
        c = lax.axis_index("c")
        s = lax.axis_index("s")
        wid = s * NC + c

        def zb(r, carry):
            for g in range(C // 16):
                zbuf[r, pl.ds(g * 16, 16)] = jnp.zeros((16,), jnp.float32)
            return carry
        lax.fori_loop(0, 104, zb, 0)
        for z in range(q // 104):
            pltpu.sync_copy(zbuf, acc_sp.at[pl.ds(s * q + z * 104, 104)])

        @pl.when(s == NS - 1)
        def _():
            pltpu.sync_copy(zbuf.at[pl.ds(0, tail)],
                            acc_sp.at[pl.ds(NS * q, tail)])
        plsc.subcore_barrier()

        for tab_hbm, eir_hbm in ((a1_hbm, e10_hbm), (a1_hbm, e11_hbm),
                                 (a2_hbm, e20_hbm), (a2_hbm, e21_hbm)):
            def body(b, carry):
                base = wid * rows_per_w + b * GB
                pltpu.sync_copy(eir_hbm.at[pl.ds(base, GB)], idx_v)
                pltpu.sync_copy(tab_hbm.at[pl.ds(base, GB)], rows_v)
                pltpu.sync_copy(rows_v, acc_sp.at[idx_v], add=True)
                return carry
            lax.fori_loop(0, nb, body, 0)

        plsc.subcore_barrier()
        pltpu.sync_copy(acc_sp.at[pl.ds(s * q, q)],
                        p_hbm.at[c, pl.ds(s * q, q)])

        @pl.when(s == NS - 1)
        def _():
            pltpu.sync_copy(acc_sp.at[pl.ds(NS * q, tail)],
                            p_hbm.at[c, pl.ds(NS * q, tail)])

    ei1 = ei1.astype(jnp.int32)
    ei2 = ei2.astype(jnp.int32)
    return k(a1, a2, ei1[0], ei1[1], ei2[0], ei2[1])


def _sc_edge_base(attr, a0, ei):
    E, C = attr.shape
    N, _ = a0.shape
    GB = 80
    rows_per_w = E // NW
    nb = rows_per_w // GB
    q, tail = _stripes(N)

    @functools.partial(
        pl.kernel, mesh=_mesh(),
        out_type=jax.ShapeDtypeStruct((E, C), jnp.float32),
        scratch_types=[
            pltpu.VMEM_SHARED((N, C), jnp.float32),
            pltpu.VMEM((GB,), jnp.int32),
            pltpu.VMEM((GB,), jnp.int32),
            pltpu.VMEM((GB, C), jnp.float32),
            pltpu.VMEM((GB, C), jnp.float32),
            pltpu.VMEM((GB, C), jnp.float32),
            pltpu.SemaphoreType.DMA,
            pltpu.SemaphoreType.DMA,
        ],
    )
    def k(attr_hbm, a0_hbm, e0_hbm, e1_hbm, out_hbm, a0_sp, i0_v, i1_v, ga_v,
          gb_v, av_v, sem0, sem1):
        c = lax.axis_index("c")
        s = lax.axis_index("s")
        wid = s * NC + c

        pltpu.sync_copy(a0_hbm.at[pl.ds(s * q, q)], a0_sp.at[pl.ds(s * q, q)])

        @pl.when(s == NS - 1)
        def _():
            pltpu.sync_copy(a0_hbm.at[pl.ds(NS * q, tail)],
                            a0_sp.at[pl.ds(NS * q, tail)])
        plsc.subcore_barrier()

        def body(b, carry):
            base = wid * rows_per_w + b * GB
            pltpu.sync_copy(e0_hbm.at[pl.ds(base, GB)], i0_v)
            pltpu.sync_copy(e1_hbm.at[pl.ds(base, GB)], i1_v)
            cp0 = pltpu.async_copy(a0_sp.at[i0_v], ga_v, sem0)
            cp1 = pltpu.async_copy(a0_sp.at[i1_v], gb_v, sem1)
            pltpu.sync_copy(attr_hbm.at[pl.ds(base, GB)], av_v)
            cp0.wait()
            cp1.wait()
            _rowmul(av_v, ga_v, gb_v, GB, C // 16, extra=av_v)
            pltpu.sync_copy(av_v, out_hbm.at[pl.ds(base, GB)])
            return carry

        lax.fori_loop(0, nb, body, 0)

    ei = ei.astype(jnp.int32)
    return k(attr, a0, ei[0], ei[1])


def _sc_tri_acc(tris_jk, tab1, tab2, base):
    E, C = base.shape
    T = tris_jk[0][0].shape[1]
    CH = E // 32
    stripe = CH // NS
    IB = 4000
    tpt = T // NS
    nib = tpt // IB
    nvs = IB // 16
    CAP = 128
    FTH = CAP - 16
    assert T % NS == 0 and (T // NS) % IB == 0 and IB % 16 == 0

    tabs = {1: 0, 2: 1}
    q, tail = _stripes(CH)

    @functools.partial(
        pl.kernel, mesh=_mesh(),
        compiler_params=pltpu.CompilerParams(needs_layout_passes=False),
        out_type=jax.ShapeDtypeStruct((E, C), jnp.float32),
        scratch_types=[
            pltpu.VMEM_SHARED((CH + 8, C), jnp.float32),
            pltpu.VMEM((3 * IB,), jnp.int32),
            pltpu.VMEM((CAP,), jnp.int32),
            pltpu.VMEM((CAP,), jnp.int32),
            pltpu.VMEM((CAP,), jnp.int32),
            pltpu.VMEM((CAP, C), jnp.float32),
            pltpu.VMEM((CAP, C), jnp.float32),
            pltpu.SemaphoreType.DMA,
            pltpu.SemaphoreType.DMA,
        ],
    )
    def k(tp0, tp1, tp2, tA_hbm, tB_hbm,
          base_hbm, out_hbm, acc_sp, isp, cb0, cb1, cb2, gj_v,
          gk_v, sem0, sem1):
        c = lax.axis_index("c")
        s = lax.axis_index("s")
        tri_hbms = (tp0, tp1, tp2)
        tab_hbms = (tA_hbm, tB_hbm)

        lanes = lax.iota(jnp.int32, 16)

        def reset_cbufs():
            for g in range(CAP // 16):
                sl = pl.ds(g * 16, 16)
                cb0[sl] = jnp.full((16,), CH, jnp.int32)
                cb1[sl] = lanes + g * 16
                cb2[sl] = lanes + g * 16

        def chunk_body(ch_i, carry0):
            chunk = c * 16 + ch_i
            lo = chunk * CH

            pltpu.sync_copy(base_hbm.at[pl.ds(lo + s * q, q)],
                            acc_sp.at[pl.ds(s * q, q)])

            @pl.when(s == NS - 1)
            def _():
                pltpu.sync_copy(base_hbm.at[pl.ds(lo + NS * q, tail)],
                                acc_sp.at[pl.ds(NS * q, tail)])
            plsc.subcore_barrier()

            for tri_idx, (tri, j, kk) in enumerate(tris_jk):
                tpk = tri_hbms[tri_idx]
                tj_hbm = tab_hbms[tabs[j]]
                tk_hbm = tab_hbms[tabs[kk]]

                def flush():
                    cpj = pltpu.async_copy(tj_hbm.at[cb1], gj_v, sem0)
                    cpk = pltpu.async_copy(tk_hbm.at[cb2], gk_v, sem1)
                    cpj.wait()
                    cpk.wait()
                    _rowmul(gj_v, gj_v, gk_v, CAP, C // 16)
                    pltpu.sync_copy(gj_v, acc_sp.at[cb0], add=True)
                    reset_cbufs()

                def batch_body(b, cnt):
                    g = s * (tpt // IB) + b
                    pltpu.sync_copy(tpk.at[pl.ds(g * 3 * IB, 3 * IB)], isp)

                    def vec_body(v, cnt):
                        sl = pl.ds(v * 16, 16)
                        t0 = isp[sl]
                        m = (t0 >= lo) & (t0 < lo + CH)
                        nm = jnp.sum(m.astype(jnp.int32))

                        def compact(cnt):
                            need = cnt > FTH
                            @pl.when(need)
                            def _():
                                flush()
                            cnt = jnp.where(need, 0, cnt)
                            mi = m.astype(jnp.int32)
                            pos = cnt + plsc.cumsum(mi) - 1
                            sl1 = pl.ds(IB + v * 16, 16)
                            sl2 = pl.ds(2 * IB + v * 16, 16)
                            plsc.store_scatter(cb0, [pos], t0 - lo, mask=m)
                            plsc.store_scatter(cb1, [pos], isp[sl1], mask=m)
                            plsc.store_scatter(cb2, [pos], isp[sl2], mask=m)
                            return cnt + nm

                        return lax.cond(nm > 0, compact, lambda cnt: cnt, cnt)

                    return lax.fori_loop(0, nvs, vec_body, cnt)

                reset_cbufs()
                cnt = lax.fori_loop(0, nib, batch_body, jnp.int32(0))

                @pl.when(cnt > 0)
                def _():
                    flush()

            plsc.subcore_barrier()
            pltpu.sync_copy(acc_sp.at[pl.ds(s * q, q)],
                            out_hbm.at[pl.ds(lo + s * q, q)])

            @pl.when(s == NS - 1)
            def _():
                pltpu.sync_copy(acc_sp.at[pl.ds(NS * q, tail)],
                                out_hbm.at[pl.ds(lo + NS * q, tail)])
            plsc.subcore_barrier()
            return carry0

        lax.fori_loop(0, 16, chunk_body, 0)

    def pack(tri):
        t = tri.astype(jnp.int32)
        return t.reshape(3, T // IB, IB).transpose(1, 0, 2).reshape(-1)

    t = [pack(x[0]) for x in tris_jk]
    return k(t[0], t[1], t[2], tab1, tab2, base)


def _blk(R):
    return 512 if R % 512 == 0 else 400


def _matmul_stats(xs, w, b, coef):
    R, C = xs[0].shape
    BLK = _blk(R)
    grid = R // BLK
    nx = len(xs)

    def body(*refs):
        xa_refs = refs[:nx]
        w_ref, b_ref, y_ref, s1_ref, s2_ref = refs[nx:]
        i = pl.program_id(0)
        x = xa_refs[0][...]
        for r in xa_refs[1:]:
            x = x + r[...]
        if coef != 1.0:
            x = x * coef
        y = jnp.dot(x, w_ref[...], preferred_element_type=jnp.float32)
        y = y + b_ref[...]
        y_ref[...] = y
        ps1 = jnp.broadcast_to(jnp.sum(y, axis=0, keepdims=True), s1_ref.shape)
        ps2 = jnp.broadcast_to(jnp.sum(y * y, axis=0, keepdims=True),
                               s2_ref.shape)

        @pl.when(i == 0)
        def _():
            s1_ref[...] = jnp.zeros_like(s1_ref)
            s2_ref[...] = jnp.zeros_like(s2_ref)

        s1_ref[...] += ps1
        s2_ref[...] += ps2

    return pl.pallas_call(
        body,
        grid=(grid,),
        in_specs=[pl.BlockSpec((BLK, C), lambda i: (i, 0)) for _ in range(nx)]
        + [pl.BlockSpec((C, C), lambda i: (0, 0)),
           pl.BlockSpec((C,), lambda i: (0,))],
        out_specs=[pl.BlockSpec((BLK, C), lambda i: (i, 0)),
                   pl.BlockSpec((8, C), lambda i: (0, 0)),
                   pl.BlockSpec((8, C), lambda i: (0, 0))],
        out_shape=[jax.ShapeDtypeStruct((R, C), jnp.float32),
                   jax.ShapeDtypeStruct((8, C), jnp.float32),
                   jax.ShapeDtypeStruct((8, C), jnp.float32)],
    )(*xs, w, b)


def _bn_relu_kernel(y_ref, s1_ref, s2_ref, g_ref, be_ref, o_ref, *, R):
    mu = s1_ref[0:1, :] / R
    var = s2_ref[0:1, :] / R - mu * mu
    o_ref[...] = jnp.maximum(
        (y_ref[...] - mu) * lax.rsqrt(var + 1e-5) * g_ref[...] + be_ref[...],
        0.0)


def _bn_relu(y, s1, s2, g, be):
    R, C = y.shape
    BLK = _blk(R)
    grid = R // BLK
    return pl.pallas_call(
        functools.partial(_bn_relu_kernel, R=float(R)),
        grid=(grid,),
        in_specs=[pl.BlockSpec((BLK, C), lambda i: (i, 0)),
                  pl.BlockSpec((8, C), lambda i: (0, 0)),
                  pl.BlockSpec((8, C), lambda i: (0, 0)),
                  pl.BlockSpec((C,), lambda i: (0,)),
                  pl.BlockSpec((C,), lambda i: (0,))],
        out_specs=pl.BlockSpec((BLK, C), lambda i: (i, 0)),
        out_shape=jax.ShapeDtypeStruct((R, C), jnp.float32),
    )(y, s1, s2, g, be)


def _mm_kernel(x_ref, w_ref, b_ref, o_ref):
    o_ref[...] = jnp.dot(x_ref[...], w_ref[...],
                         preferred_element_type=jnp.float32) + b_ref[...]


def _mm(x, w, b):
    R, C = x.shape
    BLK = _blk(R)
    grid = R // BLK
    return pl.pallas_call(
        _mm_kernel,
        grid=(grid,),
        in_specs=[pl.BlockSpec((BLK, C), lambda i: (i, 0)),
                  pl.BlockSpec((C, C), lambda i: (0, 0)),
                  pl.BlockSpec((C,), lambda i: (0,))],
        out_specs=pl.BlockSpec((BLK, C), lambda i: (i, 0)),
        out_shape=jax.ShapeDtypeStruct((R, C), jnp.float32),
    )(x, w, b)


def kernel(a0, a1, a2, ei1, ei2, tri_111, tri_112, tri_122, tri_211, tri_212,
           tri_222, inv1, inv2, W_gnn, b_gnn, gamma, beta, W_out, b_out):
    tris = {(1, 1, 1): tri_111, (1, 1, 2): tri_112, (1, 2, 2): tri_122,
            (2, 1, 1): tri_211, (2, 1, 2): tri_212, (2, 2, 2): tri_222}
    eis = [None, ei1, ei2]
    invs = [None, inv1, inv2]
    attrs = [a0, a1, a2]
    for layer in range(L):
        p = _sc_node_agg(attrs[1], attrs[2], ei1, ei2)
        hraw = [None, None, None]
        for l in (1, 2):
            base = _sc_edge_base(attrs[l], attrs[0], eis[l])
            tjk = [(tris[(l, 1, 1)], 1, 1), (tris[(l, 1, 2)], 1, 2),
                   (tris[(l, 2, 2)], 2, 2)]
            hraw[l] = _sc_tri_acc(tjk, attrs[1], attrs[2], base)
        new_attrs = []
        for l in range(3):
            if l > 0:
                hg = _sc_gather_rows(hraw[l], invs[l])
                y, s1, s2 = _matmul_stats([hraw[l], hg], W_gnn[layer, l],
                                          b_gnn[layer, l], 0.5)
            else:
                y, s1, s2 = _matmul_stats([attrs[0], p[0], p[1]], W_gnn[layer, l],
                                          b_gnn[layer, l], 1.0)
            new_attrs.append(_bn_relu(y, s1, s2, gamma[layer, l],
                                      beta[layer, l]))
        attrs = new_attrs
    return tuple(_mm(attrs[l], W_out[l], b_out[l]) for l in range(3))


# speedup vs baseline: 1.4047x; 1.0080x over previous
import functools

import jax
import jax.numpy as jnp
from jax import lax
from jax.experimental import pallas as pl
from jax.experimental.pallas import tpu as pltpu
from jax.experimental.pallas import tpu_sc as plsc

L = 2
EPS = 0.0
AGGRS = ((1, 1, 1), (1, 1, 2), (1, 2, 2), (2, 1, 1), (2, 1, 2), (2, 2, 2))

NC = 2
NS = 16
NW = NC * NS


def _mesh():
    return plsc.VectorSubcoreMesh(core_axis_name="c", subcore_axis_name="s")


def _rowmul(dst_ref, a_ref, b_ref, nrows, groups, extra=None):
    def body(r, carry):
        for g in range(groups):
            sl = pl.ds(g * 16, 16)
            v = a_ref[r, sl] * b_ref[r, sl]
            if extra is not None:
                v = v + extra[r, sl]
            dst_ref[r, sl] = v
        return carry
    lax.fori_loop(0, nrows, body, 0)


def _stripes(total):
    q = (total // NS) // 8 * 8
    return q, total - NS * q

def _sc_gather_rows(table, idx):
    R, C = table.shape
    rows_per_w = R // NW
    GB = 400
    nb = rows_per_w // GB
    assert rows_per_w % GB == 0, (R, rows_per_w)

    @functools.partial(
        pl.kernel, mesh=_mesh(),
        out_type=jax.ShapeDtypeStruct((R, C), jnp.float32),
        scratch_types=[
            pltpu.VMEM((GB,), jnp.int32),
            pltpu.VMEM((GB, C), jnp.float32),
            pltpu.SemaphoreType.DMA,
        ],
    )
    def k(table_hbm, idx_hbm, out_hbm, idx_v, rows_v, sem):
        wid = lax.axis_index("s") * NC + lax.axis_index("c")

        def body(b, carry):
            base = wid * rows_per_w + b * GB
            pltpu.sync_copy(idx_hbm.at[pl.ds(base, GB)], idx_v)
            pltpu.async_copy(table_hbm.at[idx_v], rows_v, sem).wait()
            pltpu.sync_copy(rows_v, out_hbm.at[pl.ds(base, GB)])
            return carry

        lax.fori_loop(0, nb, body, 0)

    return k(table, idx.astype(jnp.int32))


def _sc_node_agg(a1, a2, ei1, ei2):
    E, C = a1.shape
    N = 10000
    GB = 200
    rows_per_w = E // NW
    nb = rows_per_w // GB
    stripe = N // NS

    q, tail = _stripes(N)

    @functools.partial(
        pl.kernel, mesh=_mesh(),
        out_type=jax.ShapeDtypeStruct((2, N, C), jnp.float32),
        scratch_types=[
            pltpu.VMEM_SHARED((N, C), jnp.float32),
            pltpu.VMEM((GB,), jnp.int32),
            pltpu.VMEM((GB, C), jnp.float32),
            pltpu.VMEM((104, C), jnp.float32),
            pltpu.SemaphoreType.DMA,
        ],
    )
    def k(a1_hbm, a2_hbm, e10_hbm, e11_hbm, e20_hbm, e21_hbm, p_hbm, acc_sp,
          idx_v, rows_v, zbuf, sem):
        c = lax.axis_index("c")
        s = lax.axis_index("s")
        wid = s * NC + c

        def zb(r, carry):
            for g in range(C // 16):
                zbuf[r, pl.ds(g * 16, 16)] = jnp.zeros((16,), jnp.float32)
            return carry
        lax.fori_loop(0, 104, zb, 0)
        for z in range(q // 104):
            pltpu.sync_copy(zbuf, acc_sp.at[pl.ds(s * q + z * 104, 104)])

        @pl.when(s == NS - 1)
        def _():
            pltpu.sync_copy(zbuf.at[pl.ds(0, tail)],
                            acc_sp.at[pl.ds(NS * q, tail)])
        plsc.subcore_barrier()

        for tab_hbm, eir_hbm in ((a1_hbm, e10_hbm), (a1_hbm, e11_hbm),
                                 (a2_hbm, e20_hbm), (a2_hbm, e21_hbm)):
            def body(b, carry):
                base = wid * rows_per_w + b * GB
                pltpu.sync_copy(eir_hbm.at[pl.ds(base, GB)], idx_v)
                pltpu.sync_copy(tab_hbm.at[pl.ds(base, GB)], rows_v)
                pltpu.sync_copy(rows_v, acc_sp.at[idx_v], add=True)
                return carry
            lax.fori_loop(0, nb, body, 0)

        plsc.subcore_barrier()
        pltpu.sync_copy(acc_sp.at[pl.ds(s * q, q)],
                        p_hbm.at[c, pl.ds(s * q, q)])

        @pl.when(s == NS - 1)
        def _():
            pltpu.sync_copy(acc_sp.at[pl.ds(NS * q, tail)],
                            p_hbm.at[c, pl.ds(NS * q, tail)])

    ei1 = ei1.astype(jnp.int32)
    ei2 = ei2.astype(jnp.int32)
    return k(a1, a2, ei1[0], ei1[1], ei2[0], ei2[1])


def _sc_edge_base(attr, a0, ei):
    E, C = attr.shape
    N, _ = a0.shape
    GB = 80
    rows_per_w = E // NW
    nb = rows_per_w // GB
    q, tail = _stripes(N)

    @functools.partial(
        pl.kernel, mesh=_mesh(),
        out_type=jax.ShapeDtypeStruct((E, C), jnp.float32),
        scratch_types=[
            pltpu.VMEM_SHARED((N, C), jnp.float32),
            pltpu.VMEM((GB,), jnp.int32),
            pltpu.VMEM((GB,), jnp.int32),
            pltpu.VMEM((GB, C), jnp.float32),
            pltpu.VMEM((GB, C), jnp.float32),
            pltpu.VMEM((GB, C), jnp.float32),
            pltpu.SemaphoreType.DMA,
            pltpu.SemaphoreType.DMA,
        ],
    )
    def k(attr_hbm, a0_hbm, e0_hbm, e1_hbm, out_hbm, a0_sp, i0_v, i1_v, ga_v,
          gb_v, av_v, sem0, sem1):
        c = lax.axis_index("c")
        s = lax.axis_index("s")
        wid = s * NC + c

        pltpu.sync_copy(a0_hbm.at[pl.ds(s * q, q)], a0_sp.at[pl.ds(s * q, q)])

        @pl.when(s == NS - 1)
        def _():
            pltpu.sync_copy(a0_hbm.at[pl.ds(NS * q, tail)],
                            a0_sp.at[pl.ds(NS * q, tail)])
        plsc.subcore_barrier()

        def body(b, carry):
            base = wid * rows_per_w + b * GB
            pltpu.sync_copy(e0_hbm.at[pl.ds(base, GB)], i0_v)
            pltpu.sync_copy(e1_hbm.at[pl.ds(base, GB)], i1_v)
            cp0 = pltpu.async_copy(a0_sp.at[i0_v], ga_v, sem0)
            cp1 = pltpu.async_copy(a0_sp.at[i1_v], gb_v, sem1)
            pltpu.sync_copy(attr_hbm.at[pl.ds(base, GB)], av_v)
            cp0.wait()
            cp1.wait()
            _rowmul(av_v, ga_v, gb_v, GB, C // 16, extra=av_v)
            pltpu.sync_copy(av_v, out_hbm.at[pl.ds(base, GB)])
            return carry

        lax.fori_loop(0, nb, body, 0)

    ei = ei.astype(jnp.int32)
    return k(attr, a0, ei[0], ei[1])


def _sc_tri_acc(tris_jk, tab1, tab2, base):
    E, C = base.shape
    T = tris_jk[0][0].shape[1]
    CH = E // 32
    stripe = CH // NS
    IB = 2000
    tpt = T // NS
    nib = tpt // IB
    nvs = IB // 16
    CAP = 128
    FTH = CAP - 16
    assert T % NS == 0 and (T // NS) % IB == 0 and IB % 16 == 0

    tabs = {1: 0, 2: 1}
    q, tail = _stripes(CH)

    @functools.partial(
        pl.kernel, mesh=_mesh(),
        compiler_params=pltpu.CompilerParams(needs_layout_passes=False),
        out_type=jax.ShapeDtypeStruct((E, C), jnp.float32),
        scratch_types=[
            pltpu.VMEM_SHARED((CH + 8, C), jnp.float32),
            pltpu.VMEM((3 * IB,), jnp.int32),
            pltpu.VMEM((3 * IB,), jnp.int32),
            pltpu.VMEM((CAP,), jnp.int32),
            pltpu.VMEM((CAP,), jnp.int32),
            pltpu.VMEM((CAP,), jnp.int32),
            pltpu.VMEM((CAP, C), jnp.float32),
            pltpu.VMEM((CAP, C), jnp.float32),
            pltpu.SemaphoreType.DMA,
            pltpu.SemaphoreType.DMA,
            pltpu.SemaphoreType.DMA,
            pltpu.SemaphoreType.DMA,
        ],
    )
    def k(tp0, tp1, tp2, tA_hbm, tB_hbm,
          base_hbm, out_hbm, acc_sp, ispA, ispB, cb0, cb1, cb2, gj_v,
          gk_v, sem0, sem1, semA, semB):
        c = lax.axis_index("c")
        s = lax.axis_index("s")
        tri_hbms = (tp0, tp1, tp2)
        tab_hbms = (tA_hbm, tB_hbm)

        lanes = lax.iota(jnp.int32, 16)

        def reset_cbufs():
            for g in range(CAP // 16):
                sl = pl.ds(g * 16, 16)
                cb0[sl] = jnp.full((16,), CH, jnp.int32)
                cb1[sl] = lanes + g * 16
                cb2[sl] = lanes + g * 16

        def chunk_body(ch_i, carry0):
            chunk = c * 16 + ch_i
            lo = chunk * CH

            pltpu.sync_copy(base_hbm.at[pl.ds(lo + s * q, q)],
                            acc_sp.at[pl.ds(s * q, q)])

            @pl.when(s == NS - 1)
            def _():
                pltpu.sync_copy(base_hbm.at[pl.ds(lo + NS * q, tail)],
                                acc_sp.at[pl.ds(NS * q, tail)])
            plsc.subcore_barrier()

            for tri_idx, (tri, j, kk) in enumerate(tris_jk):
                tpk = tri_hbms[tri_idx]
                tj_hbm = tab_hbms[tabs[j]]
                tk_hbm = tab_hbms[tabs[kk]]

                def flush():
                    cpj = pltpu.async_copy(tj_hbm.at[cb1], gj_v, sem0)
                    cpk = pltpu.async_copy(tk_hbm.at[cb2], gk_v, sem1)
                    cpj.wait()
                    cpk.wait()
                    _rowmul(gj_v, gj_v, gk_v, CAP, C // 16)
                    pltpu.sync_copy(gj_v, acc_sp.at[cb0], add=True)
                    reset_cbufs()

                def make_vec_body(isp):
                    def vec_body(v, cnt):
                        sl = pl.ds(v * 16, 16)
                        t0 = isp[sl]
                        m = (t0 >= lo) & (t0 < lo + CH)
                        nm = jnp.sum(m.astype(jnp.int32))

                        def compact(cnt):
                            need = cnt > FTH
                            @pl.when(need)
                            def _():
                                flush()
                            cnt = jnp.where(need, 0, cnt)
                            mi = m.astype(jnp.int32)
                            pos = cnt + plsc.cumsum(mi) - 1
                            sl1 = pl.ds(IB + v * 16, 16)
                            sl2 = pl.ds(2 * IB + v * 16, 16)
                            plsc.store_scatter(cb0, [pos], t0 - lo, mask=m)
                            plsc.store_scatter(cb1, [pos], isp[sl1], mask=m)
                            plsc.store_scatter(cb2, [pos], isp[sl2], mask=m)
                            return cnt + nm

                        return lax.cond(nm > 0, compact, lambda cnt: cnt, cnt)
                    return vec_body

                def pair_body(i, cnt):
                    g = s * (tpt // IB) + 2 * i
                    cpA = pltpu.async_copy(
                        tpk.at[pl.ds(g * 3 * IB, 3 * IB)], ispA, semA)
                    cpB = pltpu.async_copy(
                        tpk.at[pl.ds((g + 1) * 3 * IB, 3 * IB)], ispB, semB)
                    cpA.wait()
                    cnt = lax.fori_loop(0, nvs, make_vec_body(ispA), cnt)
                    cpB.wait()
                    cnt = lax.fori_loop(0, nvs, make_vec_body(ispB), cnt)
                    return cnt

                reset_cbufs()
                assert nib % 2 == 0
                cnt = lax.fori_loop(0, nib // 2, pair_body, jnp.int32(0))

                @pl.when(cnt > 0)
                def _():
                    flush()

            plsc.subcore_barrier()
            pltpu.sync_copy(acc_sp.at[pl.ds(s * q, q)],
                            out_hbm.at[pl.ds(lo + s * q, q)])

            @pl.when(s == NS - 1)
            def _():
                pltpu.sync_copy(acc_sp.at[pl.ds(NS * q, tail)],
                                out_hbm.at[pl.ds(lo + NS * q, tail)])
            plsc.subcore_barrier()
            return carry0

        lax.fori_loop(0, 16, chunk_body, 0)

    def pack(tri):
        t = tri.astype(jnp.int32)
        return t.reshape(3, T // IB, IB).transpose(1, 0, 2).reshape(-1)

    t = [pack(x[0]) for x in tris_jk]
    return k(t[0], t[1], t[2], tab1, tab2, base)


def _blk(R):
    return 512 if R % 512 == 0 else 400


def _matmul_stats(xs, w, b, coef):
    R, C = xs[0].shape
    BLK = _blk(R)
    grid = R // BLK
    nx = len(xs)

    def body(*refs):
        xa_refs = refs[:nx]
        w_ref, b_ref, y_ref, s1_ref, s2_ref = refs[nx:]
        i = pl.program_id(0)
        x = xa_refs[0][...]
        for r in xa_refs[1:]:
            x = x + r[...]
        if coef != 1.0:
            x = x * coef
        y = jnp.dot(x, w_ref[...], preferred_element_type=jnp.float32)
        y = y + b_ref[...]
        y_ref[...] = y
        ps1 = jnp.broadcast_to(jnp.sum(y, axis=0, keepdims=True), s1_ref.shape)
        ps2 = jnp.broadcast_to(jnp.sum(y * y, axis=0, keepdims=True),
                               s2_ref.shape)

        @pl.when(i == 0)
        def _():
            s1_ref[...] = jnp.zeros_like(s1_ref)
            s2_ref[...] = jnp.zeros_like(s2_ref)

        s1_ref[...] += ps1
        s2_ref[...] += ps2

    return pl.pallas_call(
        body,
        grid=(grid,),
        in_specs=[pl.BlockSpec((BLK, C), lambda i: (i, 0)) for _ in range(nx)]
        + [pl.BlockSpec((C, C), lambda i: (0, 0)),
           pl.BlockSpec((C,), lambda i: (0,))],
        out_specs=[pl.BlockSpec((BLK, C), lambda i: (i, 0)),
                   pl.BlockSpec((8, C), lambda i: (0, 0)),
                   pl.BlockSpec((8, C), lambda i: (0, 0))],
        out_shape=[jax.ShapeDtypeStruct((R, C), jnp.float32),
                   jax.ShapeDtypeStruct((8, C), jnp.float32),
                   jax.ShapeDtypeStruct((8, C), jnp.float32)],
    )(*xs, w, b)


def _bn_relu_kernel(y_ref, s1_ref, s2_ref, g_ref, be_ref, o_ref, *, R):
    mu = s1_ref[0:1, :] / R
    var = s2_ref[0:1, :] / R - mu * mu
    o_ref[...] = jnp.maximum(
        (y_ref[...] - mu) * lax.rsqrt(var + 1e-5) * g_ref[...] + be_ref[...],
        0.0)


def _bn_relu(y, s1, s2, g, be):
    R, C = y.shape
    BLK = _blk(R)
    grid = R // BLK
    return pl.pallas_call(
        functools.partial(_bn_relu_kernel, R=float(R)),
        grid=(grid,),
        in_specs=[pl.BlockSpec((BLK, C), lambda i: (i, 0)),
                  pl.BlockSpec((8, C), lambda i: (0, 0)),
                  pl.BlockSpec((8, C), lambda i: (0, 0)),
                  pl.BlockSpec((C,), lambda i: (0,)),
                  pl.BlockSpec((C,), lambda i: (0,))],
        out_specs=pl.BlockSpec((BLK, C), lambda i: (i, 0)),
        out_shape=jax.ShapeDtypeStruct((R, C), jnp.float32),
    )(y, s1, s2, g, be)


def _mm_kernel(x_ref, w_ref, b_ref, o_ref):
    o_ref[...] = jnp.dot(x_ref[...], w_ref[...],
                         preferred_element_type=jnp.float32) + b_ref[...]


def _mm(x, w, b):
    R, C = x.shape
    BLK = _blk(R)
    grid = R // BLK
    return pl.pallas_call(
        _mm_kernel,
        grid=(grid,),
        in_specs=[pl.BlockSpec((BLK, C), lambda i: (i, 0)),
                  pl.BlockSpec((C, C), lambda i: (0, 0)),
                  pl.BlockSpec((C,), lambda i: (0,))],
        out_specs=pl.BlockSpec((BLK, C), lambda i: (i, 0)),
        out_shape=jax.ShapeDtypeStruct((R, C), jnp.float32),
    )(x, w, b)


def kernel(a0, a1, a2, ei1, ei2, tri_111, tri_112, tri_122, tri_211, tri_212,
           tri_222, inv1, inv2, W_gnn, b_gnn, gamma, beta, W_out, b_out):
    tris = {(1, 1, 1): tri_111, (1, 1, 2): tri_112, (1, 2, 2): tri_122,
            (2, 1, 1): tri_211, (2, 1, 2): tri_212, (2, 2, 2): tri_222}
    eis = [None, ei1, ei2]
    invs = [None, inv1, inv2]
    attrs = [a0, a1, a2]
    for layer in range(L):
        p = _sc_node_agg(attrs[1], attrs[2], ei1, ei2)
        hraw = [None, None, None]
        for l in (1, 2):
            base = _sc_edge_base(attrs[l], attrs[0], eis[l])
            tjk = [(tris[(l, 1, 1)], 1, 1), (tris[(l, 1, 2)], 1, 2),
                   (tris[(l, 2, 2)], 2, 2)]
            hraw[l] = _sc_tri_acc(tjk, attrs[1], attrs[2], base)
        new_attrs = []
        for l in range(3):
            if l > 0:
                hg = _sc_gather_rows(hraw[l], invs[l])
                y, s1, s2 = _matmul_stats([hraw[l], hg], W_gnn[layer, l],
                                          b_gnn[layer, l], 0.5)
            else:
                y, s1, s2 = _matmul_stats([attrs[0], p[0], p[1]], W_gnn[layer, l],
                                          b_gnn[layer, l], 1.0)
            new_attrs.append(_bn_relu(y, s1, s2, gamma[layer, l],
                                      beta[layer, l]))
        attrs = new_attrs
    return tuple(_mm(attrs[l], W_out[l], b_out[l]) for l in range(3))
